# baseline (speedup 1.0000x reference)
import functools

import jax
import jax.numpy as jnp
from jax import lax
from jax.experimental import pallas as pl
from jax.experimental.pallas import tpu as pltpu
from jax.experimental.pallas import tpu_sc as plsc

N_NODES = 10000
N_EDGES = 160000
IN_DIM = 32
OUT_DIM = 32
EDGE_FEAT_DIM = 16
PK = 4
FD = EDGE_FEAT_DIM * OUT_DIM

NC = 2
NS = 16
NW = NC * NS
N_HALF = 2
E_H = N_EDGES // N_HALF
SZ_BIG = [632, 624, 624, 624]
SZ_SMALL = [624, 624, 624, 624]
CHUNK_MAX = 632

_sc_mesh = functools.partial(
    plsc.VectorSubcoreMesh, core_axis_name="c", subcore_axis_name="s")
_sc_params = pltpu.CompilerParams(use_tc_tiling_on_sc=False)


@functools.partial(
    pl.kernel,
    mesh=_sc_mesh(),
    out_type=jax.ShapeDtypeStruct((E_H, IN_DIM), jnp.float32),
    scratch_types=[
        pltpu.VMEM((632,), jnp.int32),
        pltpu.VMEM((624,), jnp.int32),
        pltpu.VMEM((CHUNK_MAX, IN_DIM), jnp.float32),
        pltpu.SemaphoreType.DMA,
    ],
    compiler_params=_sc_params,
)
def _gather_rows(nf_hbm, src_hbm, out_hbm, idxb_v, idxs_v, rows_v, sem):
    wid = lax.axis_index("s") * NC + lax.axis_index("c")

    def run(base, sizes):
        off = 0
        for sz in sizes:
            idx = idxb_v if sz == 632 else idxs_v
            o = pl.multiple_of(base + off, 8)
            pltpu.sync_copy(src_hbm.at[pl.ds(o, sz)], idx)
            pltpu.async_copy(nf_hbm.at[idx],
                             rows_v.at[pl.ds(0, sz)], sem).wait()
            pltpu.sync_copy(rows_v.at[pl.ds(0, sz)], out_hbm.at[pl.ds(o, sz)])
            off += sz

    @pl.when(wid < 16)
    def _():
        run(pl.multiple_of(wid * 2504, 8), SZ_BIG)

    @pl.when(wid >= 16)
    def _():
        run(pl.multiple_of(16 * 2504 + (wid - 16) * 2496, 8), SZ_SMALL)


@functools.partial(
    pl.kernel,
    mesh=_sc_mesh(),
    out_type=[jax.ShapeDtypeStruct((NC, N_NODES, OUT_DIM), jnp.float32),
              jax.ShapeDtypeStruct((NC, N_NODES, OUT_DIM), jnp.float32)],
    scratch_types=[
        pltpu.VMEM((632,), jnp.int32),
        pltpu.VMEM((624,), jnp.int32),
        pltpu.VMEM((CHUNK_MAX, OUT_DIM), jnp.float32),
        pltpu.VMEM((CHUNK_MAX, OUT_DIM), jnp.float32),
        pltpu.VMEM_SHARED((N_NODES, OUT_DIM), jnp.float32),
        pltpu.VMEM_SHARED((N_NODES, OUT_DIM), jnp.float32),
    ],
    compiler_params=_sc_params,
)
def _scatter_add(msgs_hbm, dst_hbm, zeros_hbm, ones_hbm,
                 sum_hbm, cnt_hbm, idxb_v, idxs_v, rows_v, ones_v, acc_sh, cnt_sh):
    cid = lax.axis_index("c")
    sid = lax.axis_index("s")

    pltpu.sync_copy(ones_hbm, ones_v)

    @pl.when(sid == 0)
    def _():
        pltpu.sync_copy(zeros_hbm, acc_sh)

    @pl.when(sid == 1)
    def _():
        pltpu.sync_copy(zeros_hbm, cnt_sh)

    plsc.subcore_barrier()

    wid = sid * NC + cid

    def run(base, sizes):
        off = 0
        for sz in sizes:
            idx = idxb_v if sz == 632 else idxs_v
            o = pl.multiple_of(base + off, 8)
            pltpu.sync_copy(dst_hbm.at[pl.ds(o, sz)], idx)
            pltpu.sync_copy(msgs_hbm.at[pl.ds(o, sz)], rows_v.at[pl.ds(0, sz)])
            pltpu.sync_copy(rows_v.at[pl.ds(0, sz)],
                            acc_sh.at[idx], add=True)
            pltpu.sync_copy(ones_v.at[pl.ds(0, sz)],
                            cnt_sh.at[idx], add=True)
            off += sz

    @pl.when(wid < 16)
    def _():
        run(pl.multiple_of(wid * 2504, 8), SZ_BIG)

    @pl.when(wid >= 16)
    def _():
        run(pl.multiple_of(16 * 2504 + (wid - 16) * 2496, 8), SZ_SMALL)

    plsc.subcore_barrier()

    rows_lo = 640
    r0 = pl.multiple_of(sid * rows_lo, 8)
    last = N_NODES - (NS - 1) * rows_lo

    @pl.when(sid < NS - 1)
    def _():
        pltpu.sync_copy(acc_sh.at[pl.ds(r0, rows_lo)],
                        sum_hbm.at[cid, pl.ds(r0, rows_lo)])
        pltpu.sync_copy(cnt_sh.at[pl.ds(r0, rows_lo)],
                        cnt_hbm.at[cid, pl.ds(r0, rows_lo)])

    @pl.when(sid == NS - 1)
    def _():
        pltpu.sync_copy(acc_sh.at[pl.ds((NS - 1) * rows_lo, last)],
                        sum_hbm.at[cid, pl.ds((NS - 1) * rows_lo, last)])
        pltpu.sync_copy(cnt_sh.at[pl.ds((NS - 1) * rows_lo, last)],
                        cnt_hbm.at[cid, pl.ds((NS - 1) * rows_lo, last)])


BE = 3200
B4 = BE // PK

def _edge_mm_body(ef_ref, x_ref, w_ref, b_ref, r_ref, s_ref, out_ref):
    x = x_ref[...]
    p = lax.dot_general(x, w_ref[...], (((1,), (0,)), ((), ())),
                        preferred_element_type=jnp.float32)
    ef_exp = lax.dot_general(ef_ref[...], r_ref[...], (((1,), (0,)), ((), ())),
                             preferred_element_type=jnp.float32)
    q = p * ef_exp
    acc = lax.dot_general(q, s_ref[...], (((1,), (0,)), ((), ())),
                          preferred_element_type=jnp.float32)
    acc = acc + lax.dot_general(x, b_ref[...], (((1,), (0,)), ((), ())),
                                preferred_element_type=jnp.float32)
    out_ref[...] = acc


def _edge_matmul(h, ef4, x4, w4blk, bblk, r4, s4):
    hoff = h * (E_H // BE)
    return pl.pallas_call(
        _edge_mm_body,
        grid=(E_H // BE,),
        in_specs=[
            pl.BlockSpec((B4, PK * EDGE_FEAT_DIM), lambda i: (i + hoff, 0)),
            pl.BlockSpec((B4, PK * IN_DIM), lambda i: (i, 0)),
            pl.BlockSpec((PK * IN_DIM, PK * FD), lambda i: (0, 0)),
            pl.BlockSpec((PK * IN_DIM, PK * OUT_DIM), lambda i: (0, 0)),
            pl.BlockSpec((PK * EDGE_FEAT_DIM, PK * FD), lambda i: (0, 0)),
            pl.BlockSpec((PK * FD, PK * OUT_DIM), lambda i: (0, 0)),
        ],
        out_specs=pl.BlockSpec((B4, PK * OUT_DIM), lambda i: (i, 0)),
        out_shape=jax.ShapeDtypeStruct((E_H // PK, PK * OUT_DIM), jnp.float32),
    )(ef4, x4, w4blk, bblk, r4, s4)


def _lane_fold(v):
    return (v[:, 0 * OUT_DIM:1 * OUT_DIM] + v[:, 1 * OUT_DIM:2 * OUT_DIM]
            + v[:, 2 * OUT_DIM:3 * OUT_DIM] + v[:, 3 * OUT_DIM:4 * OUT_DIM])


def _finalize_body(s_ref, c_ref, s1_ref, c1_ref, nf_ref, wr_ref, b_ref, g_ref,
                   bt_ref, out_ref):
    summed = s_ref[0] + s_ref[1] + s1_ref[0] + s1_ref[1]
    cnt = c_ref[0] + c_ref[1] + c1_ref[0] + c1_ref[1]
    aggr = summed / jnp.maximum(cnt, 1.0)
    out = aggr + lax.dot_general(nf_ref[...], wr_ref[...],
                                 (((1,), (0,)), ((), ())),
                                 preferred_element_type=jnp.float32) + b_ref[...]
    m32 = _lane_fold(jnp.sum(out, axis=0, keepdims=True)) / N_NODES
    mean = jnp.concatenate([m32] * PK, axis=1)
    d = out - mean
    v32 = _lane_fold(jnp.sum(d * d, axis=0, keepdims=True)) / N_NODES
    var = jnp.concatenate([v32] * PK, axis=1)
    out = d * lax.rsqrt(var + 1e-5) * g_ref[...] + bt_ref[...]
    out_ref[...] = jnp.where(out >= 0, out, 0.01 * out)


def _finalize(sums4, cnts4, sums4b, cnts4b, nf4, wrblk, bias4, gamma4, beta4):
    return pl.pallas_call(
        _finalize_body,
        out_shape=jax.ShapeDtypeStruct((N_NODES // PK, PK * OUT_DIM), jnp.float32),
    )(sums4, cnts4, sums4b, cnts4b, nf4, wrblk, bias4, gamma4, beta4)


def kernel(node_feat, edge_feat, edge_index, batch_index,
           num_sampled_nodes_per_hop, num_sampled_edges_per_hop,
           W_edge_net, b_edge_net, W_root, bias, bn_gamma, bn_beta):
    src = edge_index[0].astype(jnp.int32)
    dst = edge_index[1].astype(jnp.int32)
    w4 = W_edge_net.reshape(EDGE_FEAT_DIM, IN_DIM, OUT_DIM).transpose(1, 0, 2) \
                   .reshape(IN_DIM, FD)
    bmat = b_edge_net.reshape(IN_DIM, OUT_DIM)
    f_ids = jnp.arange(FD, dtype=jnp.int32) // OUT_DIM
    rmat = (f_ids[None, :] == jnp.arange(EDGE_FEAT_DIM, dtype=jnp.int32)[:, None]
            ).astype(jnp.float32)
    o_ids = jnp.arange(FD, dtype=jnp.int32) % OUT_DIM
    smat = (o_ids[:, None] == jnp.arange(OUT_DIM, dtype=jnp.int32)[None, :]
            ).astype(jnp.float32)
    eye4 = jnp.eye(PK, dtype=jnp.float32)
    w4blk = jnp.kron(eye4, w4)
    bblk = jnp.kron(eye4, bmat)
    r4 = jnp.kron(eye4, rmat)
    s4 = jnp.kron(eye4, smat)

    zeros = jnp.zeros((N_NODES, OUT_DIM), jnp.float32)
    ones = jnp.ones((CHUNK_MAX, OUT_DIM), jnp.float32)

    ef4 = edge_feat.reshape(N_EDGES // PK, PK * EDGE_FEAT_DIM)
    parts = []
    for h in range(N_HALF):
        src_h = lax.slice(src, (h * E_H,), ((h + 1) * E_H,))
        dst_h = lax.slice(dst, (h * E_H,), ((h + 1) * E_H,))
        x_src = _gather_rows(node_feat, src_h)
        x4 = x_src.reshape(-1).reshape(E_H // PK, PK * IN_DIM)
        msgs4 = _edge_matmul(h, ef4, x4, w4blk, bblk, r4, s4)
        msgs = msgs4.reshape(E_H, OUT_DIM)
        sums, cnts = _scatter_add(msgs, dst_h, zeros, ones)
        parts.append(sums.reshape(-1).reshape(NC, N_NODES // PK, PK * OUT_DIM))
        parts.append(cnts.reshape(-1).reshape(NC, N_NODES // PK, PK * OUT_DIM))
    sums4, cnts4, sums4b, cnts4b = parts
    nf4 = node_feat.reshape(N_NODES // PK, PK * IN_DIM)
    wrblk = jnp.kron(eye4, W_root)
    bias4 = jnp.tile(bias.reshape(1, OUT_DIM), (1, PK))
    gamma4 = jnp.tile(bn_gamma.reshape(1, OUT_DIM), (1, PK))
    beta4 = jnp.tile(bn_beta.reshape(1, OUT_DIM), (1, PK))
    out4 = _finalize(sums4, cnts4, sums4b, cnts4b, nf4, wrblk, bias4, gamma4, beta4)
    out = out4.reshape(N_NODES, OUT_DIM)
    return (out, edge_index, edge_feat)


# speedup vs baseline: 1.0090x; 1.0090x over previous
import functools

import jax
import jax.numpy as jnp
from jax import lax
from jax.experimental import pallas as pl
from jax.experimental.pallas import tpu as pltpu
from jax.experimental.pallas import tpu_sc as plsc

N_NODES = 10000
N_EDGES = 160000
IN_DIM = 32
OUT_DIM = 32
EDGE_FEAT_DIM = 16
PK = 4
FD = EDGE_FEAT_DIM * OUT_DIM

NC = 2
NS = 16
NW = NC * NS
E_PER_W = N_EDGES // NW
CHUNK = 1000
N_CHUNKS = E_PER_W // CHUNK

_sc_mesh = functools.partial(
    plsc.VectorSubcoreMesh, core_axis_name="c", subcore_axis_name="s")
_sc_params = pltpu.CompilerParams(use_tc_tiling_on_sc=False)


@functools.partial(
    pl.kernel,
    mesh=_sc_mesh(),
    out_type=jax.ShapeDtypeStruct((N_EDGES, IN_DIM), jnp.float32),
    scratch_types=[
        pltpu.VMEM((CHUNK,), jnp.int32),
        pltpu.VMEM((CHUNK, IN_DIM), jnp.float32),
        pltpu.SemaphoreType.DMA,
    ],
    compiler_params=_sc_params,
)
def _gather_rows(nf_hbm, src_hbm, out_hbm, idx_v, rows_v, sem):
    wid = lax.axis_index("s") * NC + lax.axis_index("c")
    base = pl.multiple_of(wid * E_PER_W, 8)
    for i in range(N_CHUNKS):
        off = pl.multiple_of(base + i * CHUNK, 8)
        pltpu.sync_copy(src_hbm.at[pl.ds(off, CHUNK)], idx_v)
        pltpu.async_copy(nf_hbm.at[idx_v], rows_v, sem).wait()
        pltpu.sync_copy(rows_v, out_hbm.at[pl.ds(off, CHUNK)])


@functools.partial(
    pl.kernel,
    mesh=_sc_mesh(),
    out_type=[jax.ShapeDtypeStruct((NC, N_NODES, OUT_DIM), jnp.float32),
              jax.ShapeDtypeStruct((NC, N_NODES, OUT_DIM), jnp.float32)],
    scratch_types=[
        pltpu.VMEM((CHUNK,), jnp.int32),
        pltpu.VMEM((CHUNK, OUT_DIM), jnp.float32),
        pltpu.VMEM((CHUNK, OUT_DIM), jnp.float32),
        pltpu.VMEM_SHARED((N_NODES, OUT_DIM), jnp.float32),
        pltpu.VMEM_SHARED((N_NODES, OUT_DIM), jnp.float32),
    ],
    compiler_params=_sc_params,
)
def _scatter_add(msgs_hbm, dst_hbm, zeros_hbm, ones_hbm,
                 sum_hbm, cnt_hbm, idx_v, rows_v, ones_v, acc_sh, cnt_sh):
    cid = lax.axis_index("c")
    sid = lax.axis_index("s")

    pltpu.sync_copy(ones_hbm, ones_v)

    @pl.when(sid == 0)
    def _():
        pltpu.sync_copy(zeros_hbm, acc_sh)

    @pl.when(sid == 1)
    def _():
        pltpu.sync_copy(zeros_hbm, cnt_sh)

    plsc.subcore_barrier()

    wid = sid * NC + cid
    base = pl.multiple_of(wid * E_PER_W, 8)
    for i in range(N_CHUNKS):
        off = pl.multiple_of(base + i * CHUNK, 8)
        pltpu.sync_copy(dst_hbm.at[pl.ds(off, CHUNK)], idx_v)
        pltpu.sync_copy(msgs_hbm.at[pl.ds(off, CHUNK)], rows_v)
        pltpu.sync_copy(rows_v, acc_sh.at[idx_v], add=True)
        pltpu.sync_copy(ones_v, cnt_sh.at[idx_v], add=True)

    plsc.subcore_barrier()

    rows_lo = 640
    r0 = pl.multiple_of(sid * rows_lo, 8)
    last = N_NODES - (NS - 1) * rows_lo

    @pl.when(sid < NS - 1)
    def _():
        pltpu.sync_copy(acc_sh.at[pl.ds(r0, rows_lo)],
                        sum_hbm.at[cid, pl.ds(r0, rows_lo)])
        pltpu.sync_copy(cnt_sh.at[pl.ds(r0, rows_lo)],
                        cnt_hbm.at[cid, pl.ds(r0, rows_lo)])

    @pl.when(sid == NS - 1)
    def _():
        pltpu.sync_copy(acc_sh.at[pl.ds((NS - 1) * rows_lo, last)],
                        sum_hbm.at[cid, pl.ds((NS - 1) * rows_lo, last)])
        pltpu.sync_copy(cnt_sh.at[pl.ds((NS - 1) * rows_lo, last)],
                        cnt_hbm.at[cid, pl.ds((NS - 1) * rows_lo, last)])


BE = 3200
B4 = BE // PK

def _edge_mm_body(ef_ref, x_ref, w_ref, b_ref, r_ref, s_ref, out_ref):
    x = x_ref[...].reshape(B4, PK * IN_DIM)
    p = lax.dot_general(x, w_ref[...], (((1,), (0,)), ((), ())),
                        preferred_element_type=jnp.float32)
    ef_exp = lax.dot_general(ef_ref[...], r_ref[...], (((1,), (0,)), ((), ())),
                             preferred_element_type=jnp.float32)
    q = p * ef_exp
    acc = lax.dot_general(q, s_ref[...], (((1,), (0,)), ((), ())),
                          preferred_element_type=jnp.float32)
    acc = acc + lax.dot_general(x, b_ref[...], (((1,), (0,)), ((), ())),
                                preferred_element_type=jnp.float32)
    out_ref[...] = acc


def _edge_matmul(ef4, x4, w4blk, bblk, r4, s4):
    return pl.pallas_call(
        _edge_mm_body,
        grid=(N_EDGES // BE,),
        in_specs=[
            pl.BlockSpec((B4, PK * EDGE_FEAT_DIM), lambda i: (i, 0)),
            pl.BlockSpec((BE * IN_DIM,), lambda i: (i,)),
            pl.BlockSpec((PK * IN_DIM, PK * FD), lambda i: (0, 0)),
            pl.BlockSpec((PK * IN_DIM, PK * OUT_DIM), lambda i: (0, 0)),
            pl.BlockSpec((PK * EDGE_FEAT_DIM, PK * FD), lambda i: (0, 0)),
            pl.BlockSpec((PK * FD, PK * OUT_DIM), lambda i: (0, 0)),
        ],
        out_specs=pl.BlockSpec((B4, PK * OUT_DIM), lambda i: (i, 0)),
        out_shape=jax.ShapeDtypeStruct((N_EDGES // PK, PK * OUT_DIM), jnp.float32),
    )(ef4, x4, w4blk, bblk, r4, s4)


def _lane_fold(v):
    return (v[:, 0 * OUT_DIM:1 * OUT_DIM] + v[:, 1 * OUT_DIM:2 * OUT_DIM]
            + v[:, 2 * OUT_DIM:3 * OUT_DIM] + v[:, 3 * OUT_DIM:4 * OUT_DIM])


def _finalize_body(s_ref, c_ref, nf_ref, wr_ref, b_ref, g_ref, bt_ref, out_ref):
    summed = s_ref[0] + s_ref[1]
    cnt = c_ref[0] + c_ref[1]
    aggr = summed / jnp.maximum(cnt, 1.0)
    out = aggr + lax.dot_general(nf_ref[...], wr_ref[...],
                                 (((1,), (0,)), ((), ())),
                                 preferred_element_type=jnp.float32) + b_ref[...]
    m32 = _lane_fold(jnp.sum(out, axis=0, keepdims=True)) / N_NODES
    mean = jnp.concatenate([m32] * PK, axis=1)
    d = out - mean
    v32 = _lane_fold(jnp.sum(d * d, axis=0, keepdims=True)) / N_NODES
    var = jnp.concatenate([v32] * PK, axis=1)
    out = d * lax.rsqrt(var + 1e-5) * g_ref[...] + bt_ref[...]
    out_ref[...] = jnp.where(out >= 0, out, 0.01 * out)


def _finalize(sums4, cnts4, nf4, wrblk, bias4, gamma4, beta4):
    return pl.pallas_call(
        _finalize_body,
        out_shape=jax.ShapeDtypeStruct((N_NODES // PK, PK * OUT_DIM), jnp.float32),
    )(sums4, cnts4, nf4, wrblk, bias4, gamma4, beta4)


def kernel(node_feat, edge_feat, edge_index, batch_index,
           num_sampled_nodes_per_hop, num_sampled_edges_per_hop,
           W_edge_net, b_edge_net, W_root, bias, bn_gamma, bn_beta):
    src = edge_index[0].astype(jnp.int32)
    dst = edge_index[1].astype(jnp.int32)
    w4 = W_edge_net.reshape(EDGE_FEAT_DIM, IN_DIM, OUT_DIM).transpose(1, 0, 2) \
                   .reshape(IN_DIM, FD)
    bmat = b_edge_net.reshape(IN_DIM, OUT_DIM)
    f_ids = jnp.arange(FD, dtype=jnp.int32) // OUT_DIM
    rmat = (f_ids[None, :] == jnp.arange(EDGE_FEAT_DIM, dtype=jnp.int32)[:, None]
            ).astype(jnp.float32)
    o_ids = jnp.arange(FD, dtype=jnp.int32) % OUT_DIM
    smat = (o_ids[:, None] == jnp.arange(OUT_DIM, dtype=jnp.int32)[None, :]
            ).astype(jnp.float32)
    eye4 = jnp.eye(PK, dtype=jnp.float32)
    w4blk = jnp.kron(eye4, w4)
    bblk = jnp.kron(eye4, bmat)
    r4 = jnp.kron(eye4, rmat)
    s4 = jnp.kron(eye4, smat)

    zeros = jnp.zeros((N_NODES, OUT_DIM), jnp.float32)
    ones = jnp.ones((CHUNK, OUT_DIM), jnp.float32)

    x_src = _gather_rows(node_feat, src)
    x4 = x_src.reshape(-1)
    ef4 = edge_feat.reshape(N_EDGES // PK, PK * EDGE_FEAT_DIM)
    msgs4 = _edge_matmul(ef4, x4, w4blk, bblk, r4, s4)
    msgs = msgs4.reshape(N_EDGES, OUT_DIM)
    sums, cnts = _scatter_add(msgs, dst, zeros, ones)
    sums4 = sums.reshape(-1).reshape(NC, N_NODES // PK, PK * OUT_DIM)
    cnts4 = cnts.reshape(-1).reshape(NC, N_NODES // PK, PK * OUT_DIM)
    nf4 = node_feat.reshape(N_NODES // PK, PK * IN_DIM)
    wrblk = jnp.kron(eye4, W_root)
    bias4 = jnp.tile(bias.reshape(1, OUT_DIM), (1, PK))
    gamma4 = jnp.tile(bn_gamma.reshape(1, OUT_DIM), (1, PK))
    beta4 = jnp.tile(bn_beta.reshape(1, OUT_DIM), (1, PK))
    out4 = _finalize(sums4, cnts4, nf4, wrblk, bias4, gamma4, beta4)
    out = out4.reshape(N_NODES, OUT_DIM)
    return (out, edge_index, edge_feat)
